# baseline (speedup 1.0000x reference)
import functools

import jax
import jax.numpy as jnp
from jax.experimental import pallas as pl
from jax.experimental.pallas import tpu as pltpu


def _single_buffered(block_shape, index_map):
    try:
        return pl.BlockSpec(block_shape, index_map,
                            pipeline_mode=pl.Buffered(buffer_count=1))
    except Exception:
        return pl.BlockSpec(block_shape, index_map)


def _fused_kernel(xa_ref, xb_ref, w_ref, g_ref, b_ref, o_ref, *,
                  ho, wo, groups, eps, slope):
    hw = ho * wo
    cout = o_ref.shape[1]

    tdot = functools.partial(
        jax.lax.dot_general,
        dimension_numbers=(((0,), (1,)), ((), ())),
        preferred_element_type=jnp.float32)
    acc = tdot(w_ref[0], xa_ref[0, 0:hw, :])
    acc += tdot(w_ref[1], xb_ref[0, 0:hw, :])
    acc += tdot(w_ref[2], xa_ref[0, wo:wo + hw, :])
    acc += tdot(w_ref[3], xb_ref[0, wo:wo + hw, :])

    cg = cout // groups
    s1 = jnp.sum(acc, axis=1, keepdims=True)
    s2 = jnp.sum(acc * acc, axis=1, keepdims=True)
    li = jax.lax.broadcasted_iota(jnp.int32, (cout, cout), 0) // cg
    lj = jax.lax.broadcasted_iota(jnp.int32, (cout, cout), 1) // cg
    agg = (li == lj).astype(jnp.float32)
    n = float(hw * cg)
    mean = jax.lax.dot(agg, s1,
                       precision=jax.lax.Precision.HIGHEST) / n
    ex2 = jax.lax.dot(agg, s2,
                      precision=jax.lax.Precision.HIGHEST) / n
    var = jnp.maximum(ex2 - mean * mean, 0.0)
    inv = jax.lax.rsqrt(var + eps)
    scale = inv * g_ref[...]
    bias = b_ref[...] - mean * scale

    z = acc * scale + bias
    o_ref[0] = jnp.where(z >= 0.0, z, slope * z).astype(o_ref.dtype)


def kernel(x_nchw, w_hwio, gamma, beta, *, num_groups=4, eps=1e-5,
           negative_slope=0.2):
    B, Cin, H, W = x_nchw.shape
    KH, KW, wcin, Cout = w_hwio.shape
    assert (KH, KW) == (4, 4) and wcin == Cin and H % 2 == 0 and W % 2 == 0
    Ho, Wo = H // 2, W // 2
    HW = Ho * Wo
    Hh, Wh = (H + 2) // 2, (W + 2) // 2
    K4 = 4 * Cin
    in_dtype = x_nchw.dtype

    xt = jnp.transpose(x_nchw, (0, 2, 3, 1))
    xp = jnp.pad(xt, ((0, 0), (1, 1), (1, 1), (0, 0)))
    xs = (xp.reshape(B, Hh, 2, Wh, 2, Cin)
            .transpose(0, 1, 3, 2, 4, 5)
            .reshape(B, Hh, Wh, K4)
            .astype(jnp.bfloat16))
    xa = xs[:, :, 0:Wo, :].reshape(B, Hh * Wo, K4)
    xb = xs[:, :, 1:Wo + 1, :].reshape(B, Hh * Wo, K4)

    w4 = (w_hwio.reshape(2, 2, 2, 2, Cin, Cout)
                .transpose(0, 2, 1, 3, 4, 5)
                .reshape(4, K4, Cout)
                .astype(jnp.bfloat16))
    g2 = gamma.reshape(Cout, 1).astype(jnp.float32)
    b2 = beta.reshape(Cout, 1).astype(jnp.float32)

    out = pl.pallas_call(
        functools.partial(_fused_kernel, ho=Ho, wo=Wo, groups=num_groups,
                          eps=eps, slope=negative_slope),
        grid=(B,),
        in_specs=[
            pl.BlockSpec((1, Hh * Wo, K4), lambda b: (b, 0, 0)),
            pl.BlockSpec((1, Hh * Wo, K4), lambda b: (b, 0, 0)),
            _single_buffered((4, K4, Cout), lambda b: (0, 0, 0)),
            _single_buffered((Cout, 1), lambda b: (0, 0)),
            _single_buffered((Cout, 1), lambda b: (0, 0)),
        ],
        out_specs=pl.BlockSpec((1, Cout, HW), lambda b: (b, 0, 0)),
        out_shape=jax.ShapeDtypeStruct((B, Cout, HW), in_dtype),
        compiler_params=pltpu.CompilerParams(
            dimension_semantics=("parallel",),
            vmem_limit_bytes=48 * 1024 * 1024),
    )(xa, xb, w4, g2, b2)

    return out.reshape(B, Cout, Ho, Wo)


# speedup vs baseline: 2.2733x; 2.2733x over previous
import functools

import jax
import jax.numpy as jnp
from jax.experimental import pallas as pl
from jax.experimental.pallas import tpu as pltpu


def _single_buffered(block_shape, index_map):
    try:
        return pl.BlockSpec(block_shape, index_map,
                            pipeline_mode=pl.Buffered(buffer_count=1))
    except Exception:
        return pl.BlockSpec(block_shape, index_map)


def _fused_kernel(x_ref, w_ref, g_ref, b_ref, o_ref, s_ref, *,
                  ho, wo, hh, wp, cin, groups, eps, slope):
    hw = ho * wo
    cout = o_ref.shape[2]

    s_ref[...] = x_ref[0].T.reshape(hh // 2, 2 * wp, cin)

    acc = jnp.zeros((hw, cout), jnp.float32)
    for dy in range(2):
        for dx in range(2):
            for py in range(2):
                for px in range(2):
                    t = 8 * dy + 4 * dx + 2 * py + px
                    a = s_ref[:, pl.ds(wp * py + 2 * dx + px, wo, 2), :]
                    a = a.reshape((hh // 2) * wo, cin)[wo * dy:wo * dy + hw]
                    acc += jnp.dot(a.astype(jnp.bfloat16), w_ref[t],
                                   preferred_element_type=jnp.float32)

    cg = cout // groups
    s1 = jnp.sum(acc, axis=0, keepdims=True)
    s2 = jnp.sum(acc * acc, axis=0, keepdims=True)
    li = jax.lax.broadcasted_iota(jnp.int32, (cout, cout), 0) // cg
    lj = jax.lax.broadcasted_iota(jnp.int32, (cout, cout), 1) // cg
    agg = (li == lj).astype(jnp.float32)
    n = float(hw * cg)
    mean = jax.lax.dot(s1, agg,
                       precision=jax.lax.Precision.HIGHEST) / n
    ex2 = jax.lax.dot(s2, agg,
                      precision=jax.lax.Precision.HIGHEST) / n
    var = jnp.maximum(ex2 - mean * mean, 0.0)
    inv = jax.lax.rsqrt(var + eps)
    scale = inv * g_ref[...]
    bias = b_ref[...] - mean * scale

    z = acc * scale + bias
    o_ref[0] = jnp.where(z >= 0.0, z, slope * z).astype(o_ref.dtype)


def kernel(x_nchw, w_hwio, gamma, beta, *, num_groups=4, eps=1e-5,
           negative_slope=0.2):
    B, Cin, H, W = x_nchw.shape
    KH, KW, wcin, Cout = w_hwio.shape
    assert (KH, KW) == (4, 4) and wcin == Cin and H % 2 == 0 and W % 2 == 0
    Ho, Wo = H // 2, W // 2
    HW = Ho * Wo
    Hh = H + 2
    Wp = -(-(W + 2) // 8) * 8
    in_dtype = x_nchw.dtype

    xp = jnp.pad(x_nchw, ((0, 0), (0, 0), (1, 1), (1, Wp - W - 1)))
    xp = xp.reshape(B, Cin, Hh * Wp)

    w16 = (w_hwio.reshape(2, 2, 2, 2, Cin, Cout)
                 .transpose(0, 2, 1, 3, 4, 5)
                 .reshape(16, Cin, Cout)
                 .astype(jnp.bfloat16))
    g2 = gamma.reshape(1, Cout).astype(jnp.float32)
    b2 = beta.reshape(1, Cout).astype(jnp.float32)

    out = pl.pallas_call(
        functools.partial(_fused_kernel, ho=Ho, wo=Wo, hh=Hh, wp=Wp, cin=Cin,
                          groups=num_groups, eps=eps, slope=negative_slope),
        grid=(B,),
        in_specs=[
            pl.BlockSpec((1, Cin, Hh * Wp), lambda b: (b, 0, 0)),
            _single_buffered((16, Cin, Cout), lambda b: (0, 0, 0)),
            _single_buffered((1, Cout), lambda b: (0, 0)),
            _single_buffered((1, Cout), lambda b: (0, 0)),
        ],
        out_specs=pl.BlockSpec((1, HW, Cout), lambda b: (b, 0, 0)),
        out_shape=jax.ShapeDtypeStruct((B, HW, Cout), in_dtype),
        scratch_shapes=[pltpu.VMEM((Hh // 2, 2 * Wp, Cin), jnp.float32)],
        compiler_params=pltpu.CompilerParams(
            dimension_semantics=("parallel",),
            vmem_limit_bytes=48 * 1024 * 1024),
    )(xp, w16, g2, b2)

    return jnp.transpose(out.reshape(B, Ho, Wo, Cout), (0, 3, 1, 2))
